# baseline (speedup 1.0000x reference)
import functools

import jax
import jax.numpy as jnp
from jax import lax
from jax.experimental import pallas as pl
from jax.experimental.pallas import tpu as pltpu
from jax.experimental.pallas import tpu_sc as plsc

F32 = jnp.float32
BF16 = jnp.bfloat16
I32 = jnp.int32
LANE = 128
TOPK = 2


def _dot(a, b):
    return jax.lax.dot_general(a, b, (((a.ndim - 1,), (0,)), ((), ())),
                               preferred_element_type=F32)


def _gelu(v):
    return 0.5 * v * (1.0 + jax.lax.erf(v * 0.7071067811865476))


def _pre_kernel(x_ref, wt_ref, bt_ref, xe_ref):
    xe_ref[...] = _gelu(_dot(x_ref[...], wt_ref[...]) + bt_ref[...])


def _gate_kernel(xe_ref, emb_ref, wg_ref, bgp_ref, xb_ref, meta_ref):
    bm = xe_ref.shape[0]
    lanes = jax.lax.broadcasted_iota(I32, (bm, LANE), 1)
    xf = xe_ref[...] + emb_ref[0]
    xb_ref[...] = xf.astype(BF16)
    logits = _dot(xf.astype(BF16), wg_ref[...]) + bgp_ref[...]
    m = jnp.max(logits, axis=1, keepdims=True)
    ex = jnp.exp(logits - m)
    probs = ex / jnp.sum(ex, axis=1, keepdims=True)
    v1 = jnp.max(probs, axis=1, keepdims=True)
    f1 = jnp.min(jnp.where(probs == v1, lanes, LANE), axis=1, keepdims=True)
    sel1 = lanes == f1
    p2 = jnp.where(sel1, -1.0, probs)
    v2 = jnp.max(p2, axis=1, keepdims=True)
    f2 = jnp.min(jnp.where(p2 == v2, lanes, LANE), axis=1, keepdims=True)
    meta_ref[...] = (jnp.where(lanes == 0, f1.astype(F32), 0.0)
                     + jnp.where(lanes == 1, f2.astype(F32), 0.0)
                     + jnp.where(lanes == 2, v1, 0.0)
                     + jnp.where(lanes == 3, v2, 0.0))


def _sc_row_gather(table, idx, chunk=128):
    nr = idx.shape[0]
    d = table.shape[1]
    info = plsc.get_sparse_core_info()
    nw = info.num_cores * info.num_subcores
    nper = nr // nw
    chunk = min(chunk, nper)
    assert nper % chunk == 0 and nper % 8 == 0
    nchunks = nper // chunk
    mesh = plsc.VectorSubcoreMesh(core_axis_name="c", subcore_axis_name="s")

    @functools.partial(
        pl.kernel, mesh=mesh,
        out_type=jax.ShapeDtypeStruct((nr, d), table.dtype),
        scratch_types=[
            pltpu.VMEM((chunk,), I32),
            pltpu.VMEM((chunk, d), table.dtype),
            pltpu.SemaphoreType.DMA,
        ],
    )
    def g(table_hbm, idx_hbm, out_hbm, idx_v, rows_v, sem):
        wid = lax.axis_index("s") * info.num_cores + lax.axis_index("c")
        base = wid * nper
        for c in range(nchunks):
            off = base + c * chunk
            pltpu.sync_copy(idx_hbm.at[pl.ds(off, chunk)], idx_v)
            pltpu.async_copy(table_hbm.at[idx_v], rows_v, sem).wait()
            pltpu.sync_copy(rows_v, out_hbm.at[pl.ds(off, chunk)])

    return g(table, idx)


def _group_kernel(te_ref, xs_ref, w1_ref, b1_ref, w2_ref, b2_ref, y_ref):
    h = _gelu(_dot(xs_ref[...], w1_ref[0]) + b1_ref[0])
    y_ref[...] = (_dot(h.astype(BF16), w2_ref[0]) + b2_ref[0]).astype(BF16)


def _comb_kernel(ys0_ref, ys1_ref, meta_ref, xe_ref, emb_ref, g_ref,
                 be_ref, wp_ref, bp_ref, out_ref):
    w0 = meta_ref[:, 2:3]
    w1 = meta_ref[:, 3:4]
    mo = w0 * ys0_ref[...].astype(F32) + w1 * ys1_ref[...].astype(F32)
    m = jnp.mean(mo, axis=1, keepdims=True)
    v = jnp.mean((mo - m) ** 2, axis=1, keepdims=True)
    ln = (mo - m) / jnp.sqrt(v + 1e-5) * g_ref[0] + be_ref[0]
    y = _gelu(ln) + (xe_ref[...] + emb_ref[0])
    out_ref[...] = _dot(y.astype(BF16), wp_ref[0]) + bp_ref[0]


def kernel(x, W_txt, b_txt, l2_emb, cl_emb, Wg, bg, W1, b1, W2, b2,
           g_l2, be_l2, g_cl, be_cl, W_t2v, b_t2v, W_cl, b_cl):
    B, S, TD = x.shape
    H = W_txt.shape[1]
    E = Wg.shape[1]
    SD = W_t2v.shape[1]
    T = B * S
    TT = 2 * T
    N = TOPK * TT
    bm = min(1024, T)
    nrb = T // bm
    MG = min(512, T)
    ntiles = N // MG + E
    NPAD = ntiles * MG
    bmc = min(1024, T)
    nrc = T // bmc

    xf = x.reshape(T, TD).astype(BF16)

    xe = pl.pallas_call(
        _pre_kernel,
        grid=(nrb,),
        in_specs=[
            pl.BlockSpec((bm, TD), lambda i: (i, 0)),
            pl.BlockSpec((TD, H), lambda i: (0, 0)),
            pl.BlockSpec((1, H), lambda i: (0, 0)),
        ],
        out_specs=pl.BlockSpec((bm, H), lambda i: (i, 0)),
        out_shape=jax.ShapeDtypeStruct((T, H), F32),
    )(xf, W_txt.astype(BF16), b_txt.reshape(1, H))

    emb = jnp.concatenate([l2_emb.reshape(1, 1, H), cl_emb.reshape(1, 1, H)],
                          axis=0)
    wg_pad = jnp.zeros((H, LANE), F32).at[:, :E].set(Wg).astype(BF16)
    bg_pad = jnp.full((1, LANE), -1e30, F32).at[0, :E].set(bg)

    xb, meta = pl.pallas_call(
        _gate_kernel,
        grid=(2 * nrb,),
        in_specs=[
            pl.BlockSpec((bm, H), lambda i: (i % nrb, 0)),
            pl.BlockSpec((1, 1, H), lambda i: (i // nrb, 0, 0)),
            pl.BlockSpec((H, LANE), lambda i: (0, 0)),
            pl.BlockSpec((1, LANE), lambda i: (0, 0)),
        ],
        out_specs=[
            pl.BlockSpec((bm, H), lambda i: (i, 0)),
            pl.BlockSpec((bm, LANE), lambda i: (i, 0)),
        ],
        out_shape=[
            jax.ShapeDtypeStruct((TT, H), BF16),
            jax.ShapeDtypeStruct((TT, LANE), F32),
        ],
    )(xe, emb, wg_pad, bg_pad)

    i1 = meta[:, 0].astype(I32)
    i2 = meta[:, 1].astype(I32)
    ee = jnp.stack([i1, i2], axis=1).reshape(-1)
    oh = (ee[:, None] == jnp.arange(E, dtype=I32)[None, :]).astype(I32)
    ranks = jnp.cumsum(oh, axis=0) - 1
    rank = jnp.take_along_axis(ranks, ee[:, None], axis=1)[:, 0]
    counts = ranks[-1] + 1
    tcaps = (counts + MG - 1) // MG
    starts = jnp.concatenate(
        [jnp.zeros((1,), I32), jnp.cumsum(tcaps * MG)[:-1].astype(I32)])
    pos = starts[ee] + rank
    tok = jnp.repeat(jnp.arange(TT, dtype=I32), TOPK)
    row_src = jnp.zeros((NPAD,), I32).at[pos].set(tok)
    tile_expert = jnp.minimum(
        jnp.searchsorted(jnp.cumsum(tcaps), jnp.arange(ntiles), side='right'),
        E - 1).astype(I32)
    pos2 = jnp.concatenate([pos[0::2], pos[1::2]])

    def _to_i32(a):
        r, hh = a.shape
        return jax.lax.bitcast_convert_type(a.reshape(r, hh // 2, 2), I32)

    def _to_bf16(a):
        r, hw = a.shape
        return jax.lax.bitcast_convert_type(a, BF16).reshape(r, hw * 2)

    xs = _to_bf16(_sc_row_gather(_to_i32(xb), row_src))

    y = pl.pallas_call(
        _group_kernel,
        grid_spec=pltpu.PrefetchScalarGridSpec(
            num_scalar_prefetch=1,
            grid=(ntiles,),
            in_specs=[
                pl.BlockSpec((MG, H), lambda t, te: (t, 0)),
                pl.BlockSpec((1, H, H), lambda t, te: (te[t], 0, 0)),
                pl.BlockSpec((1, 1, H), lambda t, te: (te[t], 0, 0)),
                pl.BlockSpec((1, H, H), lambda t, te: (te[t], 0, 0)),
                pl.BlockSpec((1, 1, H), lambda t, te: (te[t], 0, 0)),
            ],
            out_specs=pl.BlockSpec((MG, H), lambda t, te: (t, 0)),
        ),
        out_shape=jax.ShapeDtypeStruct((NPAD, H), BF16),
        compiler_params=pltpu.CompilerParams(
            dimension_semantics=("arbitrary",)),
    )(tile_expert, xs, W1.astype(BF16), b1.reshape(E, 1, H),
      W2.astype(BF16), b2.reshape(E, 1, H))

    ys = _to_bf16(_sc_row_gather(_to_i32(y), pos2))

    g2 = jnp.concatenate([g_l2.reshape(1, 1, H), g_cl.reshape(1, 1, H)], 0)
    be2 = jnp.concatenate([be_l2.reshape(1, 1, H), be_cl.reshape(1, 1, H)], 0)
    wp = jnp.stack([W_t2v, W_cl], axis=0).astype(BF16)
    bp = jnp.concatenate([b_t2v.reshape(1, 1, SD), b_cl.reshape(1, 1, H)], 0)
    nd = TT // bmc

    out = pl.pallas_call(
        _comb_kernel,
        grid=(nd,),
        in_specs=[
            pl.BlockSpec((bmc, H), lambda t: (t, 0)),
            pl.BlockSpec((bmc, H), lambda t: (t + nd, 0)),
            pl.BlockSpec((bmc, LANE), lambda t: (t, 0)),
            pl.BlockSpec((bmc, H), lambda t: (t % nrc, 0)),
            pl.BlockSpec((1, 1, H), lambda t: (t // nrc, 0, 0)),
            pl.BlockSpec((1, 1, H), lambda t: (t // nrc, 0, 0)),
            pl.BlockSpec((1, 1, H), lambda t: (t // nrc, 0, 0)),
            pl.BlockSpec((1, H, H), lambda t: (t // nrc, 0, 0)),
            pl.BlockSpec((1, 1, H), lambda t: (t // nrc, 0, 0)),
        ],
        out_specs=pl.BlockSpec((bmc, H), lambda t: (t, 0)),
        out_shape=jax.ShapeDtypeStruct((TT, H), F32),
    )(ys, ys, meta, xe, emb, g2, be2, wp, bp)

    return (out[:T].reshape(B, S, SD), out[T:].reshape(B, S, H))


# speedup vs baseline: 3.9730x; 3.9730x over previous
import functools

import jax
import jax.numpy as jnp
from jax.experimental import pallas as pl
from jax.experimental.pallas import tpu as pltpu

F32 = jnp.float32
BF16 = jnp.bfloat16
LANE = 128


def _dot(a, b, precision=None):
    return jax.lax.dot_general(a, b, (((a.ndim - 1,), (0,)), ((), ())),
                               precision=precision, preferred_element_type=F32)


def _gelu(v):
    return 0.5 * v * (1.0 + jax.lax.erf(v * 0.7071067811865476))


def _pre_kernel(x_ref, wt_ref, bt_ref, xe_ref):
    xe = _dot(x_ref[...], wt_ref[...])
    xe_ref[...] = _gelu(xe + bt_ref[...])


def _gate_kernel(xe_ref, emb_ref, wg_ref, bgp_ref, xb_ref, wd_ref):
    bm = xe_ref.shape[0]
    lanes = jax.lax.broadcasted_iota(jnp.int32, (bm, LANE), 1)
    xf = xe_ref[...] + emb_ref[0]
    xb_ref[...] = xf.astype(BF16)
    logits = _dot(xf.astype(BF16), wg_ref[...]) + bgp_ref[...]
    m = jnp.max(logits, axis=1, keepdims=True)
    ex = jnp.exp(logits - m)
    probs = ex / jnp.sum(ex, axis=1, keepdims=True)
    v1 = jnp.max(probs, axis=1, keepdims=True)
    f1 = jnp.min(jnp.where(probs == v1, lanes, LANE), axis=1, keepdims=True)
    sel1 = lanes == f1
    p2 = jnp.where(sel1, -1.0, probs)
    v2 = jnp.max(p2, axis=1, keepdims=True)
    f2 = jnp.min(jnp.where(p2 == v2, lanes, LANE), axis=1, keepdims=True)
    sel2 = lanes == f2
    wd_ref[...] = jnp.where(sel1, v1, 0.0) + jnp.where(sel2, v2, 0.0)


def _moe_kernel(xb_ref, wd_ref, w1_ref, b1_ref, w2_ref, b2_ref, out_ref):
    e = pl.program_id(1)
    bm = xb_ref.shape[0]
    lanes = jax.lax.broadcasted_iota(jnp.int32, (bm, LANE), 1)
    w_col = jnp.sum(wd_ref[...] * jnp.where(lanes == e, 1.0, 0.0), axis=1,
                    keepdims=True)
    h = _gelu(_dot(xb_ref[...], w1_ref[0]) + b1_ref[0])
    wh = (h * w_col).astype(BF16)
    acc = _dot(wh, w2_ref[0]) + w_col * b2_ref[0]

    @pl.when(e == 0)
    def _init():
        out_ref[...] = acc

    @pl.when(e != 0)
    def _acc():
        out_ref[...] += acc


def _post_kernel(moe_ref, xe_ref, emb_ref, g_ref, be_ref, wp_ref, bp_ref,
                 out_ref):
    mo = moe_ref[...]
    m = jnp.mean(mo, axis=1, keepdims=True)
    v = jnp.mean((mo - m) ** 2, axis=1, keepdims=True)
    ln = (mo - m) / jnp.sqrt(v + 1e-5) * g_ref[0] + be_ref[0]
    y = _gelu(ln) + (xe_ref[...] + emb_ref[0])
    out_ref[...] = _dot(y.astype(BF16), wp_ref[0]) + bp_ref[0]


def kernel(x, W_txt, b_txt, l2_emb, cl_emb, Wg, bg, W1, b1, W2, b2,
           g_l2, be_l2, g_cl, be_cl, W_t2v, b_t2v, W_cl, b_cl):
    B, S, TD = x.shape
    H = W_txt.shape[1]
    E = Wg.shape[1]
    SD = W_t2v.shape[1]
    T = B * S
    bm = min(1024, T)
    nrb = T // bm
    nb = 2 * nrb

    xf = x.reshape(T, TD).astype(BF16)

    xe = pl.pallas_call(
        _pre_kernel,
        grid=(nrb,),
        in_specs=[
            pl.BlockSpec((bm, TD), lambda i: (i, 0)),
            pl.BlockSpec((TD, H), lambda i: (0, 0)),
            pl.BlockSpec((1, H), lambda i: (0, 0)),
        ],
        out_specs=pl.BlockSpec((bm, H), lambda i: (i, 0)),
        out_shape=jax.ShapeDtypeStruct((T, H), F32),
    )(xf, W_txt.astype(BF16), b_txt.reshape(1, H))

    emb = jnp.concatenate([l2_emb.reshape(1, 1, H), cl_emb.reshape(1, 1, H)],
                          axis=0)
    wg_pad = jnp.zeros((H, LANE), F32).at[:, :E].set(Wg).astype(BF16)
    bg_pad = jnp.full((1, LANE), -1e30, F32).at[0, :E].set(bg)
    w1b = W1.astype(BF16)
    w2b = W2.astype(BF16)

    xb, wd = pl.pallas_call(
        _gate_kernel,
        grid=(nb,),
        in_specs=[
            pl.BlockSpec((bm, H), lambda i: (i % nrb, 0)),
            pl.BlockSpec((1, 1, H), lambda i: (i // nrb, 0, 0)),
            pl.BlockSpec((H, LANE), lambda i: (0, 0)),
            pl.BlockSpec((1, LANE), lambda i: (0, 0)),
        ],
        out_specs=[
            pl.BlockSpec((bm, H), lambda i: (i, 0)),
            pl.BlockSpec((bm, LANE), lambda i: (i, 0)),
        ],
        out_shape=[
            jax.ShapeDtypeStruct((2 * T, H), BF16),
            jax.ShapeDtypeStruct((2 * T, LANE), F32),
        ],
    )(xe, emb, wg_pad, bg_pad)

    moe = pl.pallas_call(
        _moe_kernel,
        grid=(nb, E),
        in_specs=[
            pl.BlockSpec((bm, H), lambda i, e: (i, 0)),
            pl.BlockSpec((bm, LANE), lambda i, e: (i, 0)),
            pl.BlockSpec((1, H, H), lambda i, e: (e, 0, 0)),
            pl.BlockSpec((1, 1, H), lambda i, e: (e, 0, 0)),
            pl.BlockSpec((1, H, H), lambda i, e: (e, 0, 0)),
            pl.BlockSpec((1, 1, H), lambda i, e: (e, 0, 0)),
        ],
        out_specs=pl.BlockSpec((bm, H), lambda i, e: (i, 0)),
        out_shape=jax.ShapeDtypeStruct((2 * T, H), F32),
        compiler_params=pltpu.CompilerParams(
            dimension_semantics=("arbitrary", "arbitrary")),
    )(xb, wd, w1b, b1.reshape(E, 1, H), w2b, b2.reshape(E, 1, H))

    g2 = jnp.concatenate([g_l2.reshape(1, 1, H), g_cl.reshape(1, 1, H)], 0)
    be2 = jnp.concatenate([be_l2.reshape(1, 1, H), be_cl.reshape(1, 1, H)], 0)
    wp = jnp.stack([W_t2v, W_cl], axis=0).astype(BF16)
    bp = jnp.concatenate([b_t2v.reshape(1, 1, SD), b_cl.reshape(1, 1, H)], 0)

    out = pl.pallas_call(
        _post_kernel,
        grid=(nb,),
        in_specs=[
            pl.BlockSpec((bm, H), lambda i: (i, 0)),
            pl.BlockSpec((bm, H), lambda i: (i % nrb, 0)),
            pl.BlockSpec((1, 1, H), lambda i: (i // nrb, 0, 0)),
            pl.BlockSpec((1, 1, H), lambda i: (i // nrb, 0, 0)),
            pl.BlockSpec((1, 1, H), lambda i: (i // nrb, 0, 0)),
            pl.BlockSpec((1, H, H), lambda i: (i // nrb, 0, 0)),
            pl.BlockSpec((1, 1, H), lambda i: (i // nrb, 0, 0)),
        ],
        out_specs=pl.BlockSpec((bm, H), lambda i: (i, 0)),
        out_shape=jax.ShapeDtypeStruct((2 * T, H), F32),
    )(moe, xe, emb, g2, be2, wp, bp)

    return (out[:T].reshape(B, S, SD), out[T:].reshape(B, S, H))
